# Initial kernel scaffold; baseline (speedup 1.0000x reference)
import functools

import jax
import jax.numpy as jnp
from jax import lax
from jax.experimental import pallas as pl
from jax.experimental.pallas import tpu as pltpu
from jax.experimental.pallas import tpu_sc as plsc

VOCAB = 1000000
EMBED = 32
B = 4096
S = 200

NC = 2
NS = 16
NW = NC * NS

N = B * S
PER_W = N // NW
CHUNK = 128
NCH = PER_W // CHUNK
NBUF = 8
NOUT = NCH // NBUF

_mesh = plsc.VectorSubcoreMesh(core_axis_name="c", subcore_axis_name="s")


@functools.partial(
    pl.kernel,
    mesh=_mesh,
    out_type=jax.ShapeDtypeStruct((N, EMBED), jnp.float32),
    scratch_types=[
        pltpu.VMEM((NCH, CHUNK), jnp.int32),
        pltpu.VMEM((NBUF, CHUNK, EMBED), jnp.float32),
        pltpu.SemaphoreType.DMA((NBUF,)),
        pltpu.SemaphoreType.DMA((NBUF,)),
    ],
)
def _emb_lookup(idx_hbm, table_hbm, out_hbm, idx_v, rows_v, gsem, ssem):
    wid = lax.axis_index("s") * NC + lax.axis_index("c")
    base = wid * PER_W

    pltpu.sync_copy(idx_hbm.at[wid], idx_v)

    def gather_start(j, b):
        pltpu.async_copy(table_hbm.at[idx_v.at[j]], rows_v.at[b], gsem.at[b])

    def gather_wait(j, b):
        pltpu.make_async_copy(
            table_hbm.at[idx_v.at[j]], rows_v.at[b], gsem.at[b]
        ).wait()

    def store_start(j, b):
        pltpu.async_copy(
            rows_v.at[b], out_hbm.at[pl.ds(base + j * CHUNK, CHUNK)], ssem.at[b]
        )

    def store_wait(j, b):
        pltpu.make_async_copy(
            rows_v.at[b], out_hbm.at[pl.ds(base + j * CHUNK, CHUNK)], ssem.at[b]
        ).wait()

    for b in range(NBUF):
        gather_start(b, b)

    def outer(g, carry):
        for b in range(NBUF):
            j = g * NBUF + b
            gather_wait(j, b)
            store_start(j, b)
            store_wait(j, b)
            gather_start(j + NBUF, b)
        return carry

    lax.fori_loop(0, NOUT - 1, outer, 0)

    for b in range(NBUF):
        j = (NOUT - 1) * NBUF + b
        gather_wait(j, b)
        store_start(j, b)
        store_wait(j, b)


def kernel(x, table):
    idx = x.reshape(NW, NCH, CHUNK).astype(jnp.int32)
    out = _emb_lookup(idx, table)
    return out.reshape(B, S, EMBED)


# speedup vs baseline: 1.5019x; 1.5019x over previous
import functools

import jax
import jax.numpy as jnp
from jax import lax
from jax.experimental import pallas as pl
from jax.experimental.pallas import tpu as pltpu
from jax.experimental.pallas import tpu_sc as plsc

VOCAB = 1000000
EMBED = 32
B = 4096
S = 200

NC = 2
NS = 16
NW = NC * NS

N = B * S
PER_W = N // NW
CHUNK = 128
NCH = PER_W // CHUNK
NBUF = 8
NOUT = NCH // NBUF

_mesh = plsc.VectorSubcoreMesh(core_axis_name="c", subcore_axis_name="s")


@functools.partial(
    pl.kernel,
    mesh=_mesh,
    out_type=jax.ShapeDtypeStruct((N, EMBED), jnp.float32),
    scratch_types=[
        pltpu.VMEM((NCH, CHUNK), jnp.int32),
        pltpu.VMEM((NBUF, CHUNK, EMBED), jnp.float32),
        pltpu.SemaphoreType.DMA((NBUF,)),
        pltpu.SemaphoreType.DMA((NBUF,)),
    ],
    compiler_params=pltpu.CompilerParams(use_tc_tiling_on_sc=False),
)
def _emb_lookup(idx_hbm, table_hbm, out_hbm, idx_v, rows_v, gsem, ssem):
    wid = lax.axis_index("s") * NC + lax.axis_index("c")
    base = wid * PER_W

    pltpu.sync_copy(idx_hbm.at[wid], idx_v)

    def gather_start(j, b):
        pltpu.async_copy(table_hbm.at[idx_v.at[j]], rows_v.at[b], gsem.at[b])

    def gather_wait(j, b):
        pltpu.make_async_copy(
            table_hbm.at[idx_v.at[j]], rows_v.at[b], gsem.at[b]
        ).wait()

    def store_start(j, b):
        pltpu.async_copy(
            rows_v.at[b], out_hbm.at[pl.ds(base + j * CHUNK, CHUNK)], ssem.at[b]
        )

    def store_wait(j, b):
        pltpu.make_async_copy(
            rows_v.at[b], out_hbm.at[pl.ds(base + j * CHUNK, CHUNK)], ssem.at[b]
        ).wait()

    for b in range(NBUF):
        gather_start(b, b)

    def outer(g, carry):
        for b in range(NBUF):
            j = g * NBUF + b
            gather_wait(j, b)
            store_start(j, b)
            store_wait(j, b)
            gather_start(j + NBUF, b)
        return carry

    lax.fori_loop(0, NOUT - 1, outer, 0)

    for b in range(NBUF):
        j = (NOUT - 1) * NBUF + b
        gather_wait(j, b)
        store_start(j, b)
        store_wait(j, b)


def kernel(x, table):
    idx = x.reshape(NW, NCH, CHUNK).astype(jnp.int32)
    out = _emb_lookup(idx, table)
    return out.reshape(B, S, EMBED)
